# baseline (speedup 1.0000x reference)
import functools
import jax
import jax.numpy as jnp
from jax import lax
from jax.experimental import pallas as pl
from jax.experimental.pallas import tpu as pltpu
from jax.experimental.pallas import tpu_sc as plsc

_WIN = 11
_PAD = _WIN // 2
_NBIN = 1000
_NBINP = 1024
_B = 256
_H = 192
_W = 256
_ROWLEN = _H * _W
_NC = 2
_NS = 16
_NW = _NC * _NS
_ROWS_PER_W = _B // _NW
_NSLICE = _ROWLEN // 16
_UNROLL = 8


def _sc_hist_kernel(x1_hbm, x2_hbm, hist_hbm, buf0, buf1, priv, outbuf,
                    sem0, sem1):
    c = lax.axis_index("c")
    s = lax.axis_index("s")
    wid = s * _NC + c
    base = wid * _ROWS_PER_W
    lanes = lax.iota(jnp.int32, 16)
    lane_slots = lanes * 16
    zeros16 = jnp.zeros((16,), jnp.float32)
    ones16 = jnp.ones((16,), jnp.float32)

    @plsc.parallel_loop(0, 16 * _NBINP // 16, step=1, unroll=8)
    def zbody(i):
        priv[pl.ds(i * 16, 16)] = zeros16

    def compute_row(rowbuf, out_row):
        big = jnp.full((16,), 3.4e38, jnp.float32)

        @plsc.parallel_loop(0, _H, step=1, unroll=4, carry=(big, -big))
        def mmloop(i, carry):
            mn, mx = carry
            xs = [rowbuf[i, pl.ds(k * 16, 16)] for k in range(_W // 16)]
            while len(xs) > 1:
                xs = [(jnp.minimum(a[0], b[0]), jnp.maximum(a[1], b[1]))
                      if isinstance(a, tuple) else
                      (jnp.minimum(a, b), jnp.maximum(a, b))
                      for a, b in zip(xs[0::2], xs[1::2])]
            tmn, tmx = xs[0]
            return jnp.minimum(mn, tmn), jnp.maximum(mx, tmx)
        mn16, mx16 = mmloop
        for k in (1, 2, 4, 8):
            outbuf[pl.ds(0, 16)] = mn16
            outbuf[pl.ds(16, 16)] = mx16
            perm = jnp.bitwise_xor(lanes, k)
            mn16 = jnp.minimum(mn16, plsc.load_gather(outbuf.at[pl.ds(0, 16)], [perm]))
            mx16 = jnp.maximum(mx16, plsc.load_gather(outbuf.at[pl.ds(16, 16)], [perm]))
        mn = mn16
        scale = jnp.where(mx16 > mn16, mx16 - mn16, jnp.float32(1.0))
        inv = jnp.float32(_NBIN) / scale

        off = mn * inv
        @plsc.parallel_loop(0, _H, step=1, unroll=4)
        def sbody(i):
            for k in range(_W // 16):
                x = rowbuf[i, pl.ds(k * 16, 16)]
                r = x * inv - off
                idx = r.astype(jnp.int32)
                plsc.addupdate_scatter(
                    priv, [jnp.bitwise_or(idx * 16, lanes)], ones16)

        @plsc.parallel_loop(0, _NBINP // 16, step=1, unroll=2)
        def rbody(j):
            acc = zeros16
            cbase = j * 256 + lane_slots
            for s_ in range(16):
                ix = cbase + jnp.bitwise_and(lanes + s_, 15)
                acc = acc + plsc.load_gather(priv, [ix])
                plsc.store_scatter(priv, [ix], zeros16)
            outbuf[pl.ds(j * 16, 16)] = acc

        pltpu.sync_copy(outbuf, hist_hbm.at[out_row])

    def image_loop(x_hbm, out_base):
        pltpu.make_async_copy(x_hbm.at[base, 0], buf0, sem0).start()

        def pair(p, _):
            r0 = base + 2 * p
            pltpu.make_async_copy(x_hbm.at[r0 + 1, 0], buf1, sem1).start()
            pltpu.make_async_copy(x_hbm.at[r0, 0], buf0, sem0).wait()
            compute_row(buf0, out_base + r0)

            @pl.when(p < _ROWS_PER_W // 2 - 1)
            def _():
                pltpu.make_async_copy(x_hbm.at[r0 + 2, 0], buf0, sem0).start()
            pltpu.make_async_copy(x_hbm.at[r0 + 1, 0], buf1, sem1).wait()
            compute_row(buf1, out_base + r0 + 1)
            return 0
        lax.fori_loop(0, _ROWS_PER_W // 2, pair, 0)

    image_loop(x1_hbm, 0)
    image_loop(x2_hbm, _B)


def _sc_hist(x1, x2):
    mesh = plsc.VectorSubcoreMesh(core_axis_name="c", subcore_axis_name="s")
    f = functools.partial(
        pl.kernel,
        out_type=jax.ShapeDtypeStruct((2 * _B, _NBINP), jnp.float32),
        mesh=mesh,
        scratch_types=[
            pltpu.VMEM((_H, _W), jnp.float32),
            pltpu.VMEM((_H, _W), jnp.float32),
            pltpu.VMEM((16 * _NBINP,), jnp.float32),
            pltpu.VMEM((_NBINP,), jnp.float32),
            pltpu.SemaphoreType.DMA,
            pltpu.SemaphoreType.DMA,
        ],
        compiler_params=pltpu.CompilerParams(needs_layout_passes=False),
    )(_sc_hist_kernel)
    return f(x1, x2)


_NB = 4

def _ssim_kernel(x1_ref, x2_ref, m192_ref, m256t_ref, out_ref):
    @pl.when(pl.program_id(0) == 0)
    def _():
        out_ref[...] = jnp.zeros((1, 1), jnp.float32)

    m192 = m192_ref[...]
    m256t = m256t_ref[...]
    c1 = jnp.float32(0.01 ** 2)
    c2 = jnp.float32(0.03 ** 2)

    part = jnp.float32(0.0)
    for b in range(_NB):
        x1 = x1_ref[b]
        x2 = x2_ref[b]
        fields = jnp.concatenate(
            [x1, x2, x1 * x1, x2 * x2, x1 * x2], axis=1)
        v = jnp.dot(m192, fields, preferred_element_type=jnp.float32,
                    precision=lax.Precision.DEFAULT)
        convs = [
            jnp.dot(v[:, i * _W:(i + 1) * _W], m256t,
                    preferred_element_type=jnp.float32,
                    precision=lax.Precision.DEFAULT)
            for i in range(5)
        ]
        mu1, mu2, s1, s2, s12 = convs
        mu1_sq = mu1 * mu1
        mu2_sq = mu2 * mu2
        mu1_mu2 = mu1 * mu2
        sigma1_sq = s1 - mu1_sq
        sigma2_sq = s2 - mu2_sq
        sigma12 = s12 - mu1_mu2
        ssim_map = ((2 * mu1_mu2 + c1) * (2 * sigma12 + c2) /
                    ((mu1_sq + mu2_sq + c1) * (sigma1_sq + sigma2_sq + c2)))
        part = part + jnp.sum(ssim_map)

    out_ref[...] += jnp.full((1, 1), part, jnp.float32)


def _ssim_sum(x1, x2, m192, m256t):
    grid = _B // _NB
    return pl.pallas_call(
        _ssim_kernel,
        grid=(grid,),
        in_specs=[
            pl.BlockSpec((_NB, _H, _W), lambda i: (i, 0, 0)),
            pl.BlockSpec((_NB, _H, _W), lambda i: (i, 0, 0)),
            pl.BlockSpec((_H, _H), lambda i: (0, 0)),
            pl.BlockSpec((_W, _W), lambda i: (0, 0)),
        ],
        out_specs=pl.BlockSpec((1, 1), lambda i: (0, 0)),
        out_shape=jax.ShapeDtypeStruct((1, 1), jnp.float32),
        compiler_params=pltpu.CompilerParams(
            dimension_semantics=("arbitrary",)),
    )(x1, x2, m192, m256t)


def _final_kernel(hist_ref, ssim_ref, out_ref):
    cols = lax.broadcasted_iota(jnp.int32, (_B, _NBINP), 1)
    mask = cols < _NBIN
    neg = jnp.float32(-3.4e38)

    def absorb(h):
        rolled = jnp.concatenate(
            [h[:, 1:], jnp.zeros((_B, 1), jnp.float32)], axis=1)
        return h + jnp.where(cols == _NBIN - 1, rolled, 0.0)

    h1 = jnp.where(mask, absorb(hist_ref[:_B]), neg)
    h2 = jnp.where(mask, absorb(hist_ref[_B:]), neg)

    def softmax(h):
        m = jnp.max(h, axis=1, keepdims=True)
        e = jnp.exp(h - m)
        return e / jnp.sum(e, axis=1, keepdims=True)

    p1 = softmax(h1)
    p2 = softmax(h2)
    kld = jnp.sum(jnp.where(mask, jnp.exp(p2) * (p2 - p1), 0.0)) / _B

    ssim = ssim_ref[0, 0] / jnp.float32(_B * _H * _W)
    res = jnp.where(ssim > 0.75, kld + 1.0 - ssim, 1.0 - ssim)
    out_ref[...] = jnp.full((1, 1), res, jnp.float32)


def _final(hist, ssim_sum):
    return pl.pallas_call(
        _final_kernel,
        out_shape=jax.ShapeDtypeStruct((1, 1), jnp.float32),
    )(hist, ssim_sum)


def _banded(n, g):
    i = jnp.arange(n)[:, None]
    j = jnp.arange(n)[None, :]
    d = j - i + _PAD
    valid = (d >= 0) & (d <= _WIN - 1)
    return jnp.where(valid, g[jnp.clip(d, 0, _WIN - 1)], 0.0).astype(jnp.float32)


def kernel(img1, img2, window):
    x1 = img1.reshape(_B, _H, _W)
    x2 = img2.reshape(_B, _H, _W)

    w2 = window[0, 0]
    g = w2[_PAD] / jnp.sqrt(w2[_PAD, _PAD])
    m192 = _banded(_H, g)
    m256t = _banded(_W, g).T

    hist = _sc_hist(img1, img2)
    ssim_sum = _ssim_sum(x1, x2, m192, m256t)
    out = _final(hist, ssim_sum)
    return out[0, 0]


# speedup vs baseline: 1.0246x; 1.0246x over previous
import functools
import jax
import jax.numpy as jnp
from jax import lax
from jax.experimental import pallas as pl
from jax.experimental.pallas import tpu as pltpu
from jax.experimental.pallas import tpu_sc as plsc

_WIN = 11
_PAD = _WIN // 2
_NBIN = 1000
_NBINP = 1024
_B = 256
_H = 192
_W = 256
_ROWLEN = _H * _W
_NC = 2
_NS = 16
_NW = _NC * _NS
_ROWS_PER_W = _B // _NW
_NSLICE = _ROWLEN // 16
_UNROLL = 8


def _sc_hist_kernel(x1_hbm, x2_hbm, hist_hbm, buf0, buf1, priv, outbuf,
                    sem0, sem1):
    c = lax.axis_index("c")
    s = lax.axis_index("s")
    wid = s * _NC + c
    base = wid * _ROWS_PER_W
    lanes = lax.iota(jnp.int32, 16)
    lane_slots = lanes * 16
    zeros16 = jnp.zeros((16,), jnp.float32)
    ones16 = jnp.ones((16,), jnp.float32)

    @plsc.parallel_loop(0, 16 * _NBINP // 16, step=1, unroll=8)
    def zbody(i):
        priv[pl.ds(i * 16, 16)] = zeros16

    def compute_row(rowbuf, out_row):
        big = jnp.full((16,), 3.4e38, jnp.float32)

        @plsc.parallel_loop(0, _H, step=1, unroll=2, carry=(big, -big))
        def mmloop(i, carry):
            mn, mx = carry
            xs = [rowbuf[i, pl.ds(k * 16, 16)] for k in range(_W // 16)]
            while len(xs) > 1:
                xs = [(jnp.minimum(a[0], b[0]), jnp.maximum(a[1], b[1]))
                      if isinstance(a, tuple) else
                      (jnp.minimum(a, b), jnp.maximum(a, b))
                      for a, b in zip(xs[0::2], xs[1::2])]
            tmn, tmx = xs[0]
            return jnp.minimum(mn, tmn), jnp.maximum(mx, tmx)
        mn16, mx16 = mmloop
        for k in (1, 2, 4, 8):
            outbuf[pl.ds(0, 16)] = mn16
            outbuf[pl.ds(16, 16)] = mx16
            perm = jnp.bitwise_xor(lanes, k)
            mn16 = jnp.minimum(mn16, plsc.load_gather(outbuf.at[pl.ds(0, 16)], [perm]))
            mx16 = jnp.maximum(mx16, plsc.load_gather(outbuf.at[pl.ds(16, 16)], [perm]))
        mn = mn16
        scale = jnp.where(mx16 > mn16, mx16 - mn16, jnp.float32(1.0))
        inv = jnp.float32(_NBIN) / scale

        off = mn * inv
        @plsc.parallel_loop(0, _H, step=1, unroll=2)
        def sbody(i):
            for k in range(_W // 16):
                x = rowbuf[i, pl.ds(k * 16, 16)]
                r = x * inv - off
                idx = r.astype(jnp.int32)
                plsc.addupdate_scatter(
                    priv, [jnp.bitwise_or(idx * 16, lanes)], ones16)

        @plsc.parallel_loop(0, _NBINP // 16, step=1, unroll=2)
        def rbody(j):
            acc = zeros16
            cbase = j * 256 + lane_slots
            for s_ in range(16):
                ix = cbase + jnp.bitwise_and(lanes + s_, 15)
                acc = acc + plsc.load_gather(priv, [ix])
                plsc.store_scatter(priv, [ix], zeros16)
            outbuf[pl.ds(j * 16, 16)] = acc

        pltpu.sync_copy(outbuf, hist_hbm.at[out_row])

    def image_loop(x_hbm, out_base):
        pltpu.make_async_copy(x_hbm.at[base, 0], buf0, sem0).start()

        def pair(p, _):
            r0 = base + 2 * p
            pltpu.make_async_copy(x_hbm.at[r0 + 1, 0], buf1, sem1).start()
            pltpu.make_async_copy(x_hbm.at[r0, 0], buf0, sem0).wait()
            compute_row(buf0, out_base + r0)

            @pl.when(p < _ROWS_PER_W // 2 - 1)
            def _():
                pltpu.make_async_copy(x_hbm.at[r0 + 2, 0], buf0, sem0).start()
            pltpu.make_async_copy(x_hbm.at[r0 + 1, 0], buf1, sem1).wait()
            compute_row(buf1, out_base + r0 + 1)
            return 0
        lax.fori_loop(0, _ROWS_PER_W // 2, pair, 0)

    image_loop(x1_hbm, 0)
    image_loop(x2_hbm, _B)


def _sc_hist(x1, x2):
    mesh = plsc.VectorSubcoreMesh(core_axis_name="c", subcore_axis_name="s")
    f = functools.partial(
        pl.kernel,
        out_type=jax.ShapeDtypeStruct((2 * _B, _NBINP), jnp.float32),
        mesh=mesh,
        scratch_types=[
            pltpu.VMEM((_H, _W), jnp.float32),
            pltpu.VMEM((_H, _W), jnp.float32),
            pltpu.VMEM((16 * _NBINP,), jnp.float32),
            pltpu.VMEM((_NBINP,), jnp.float32),
            pltpu.SemaphoreType.DMA,
            pltpu.SemaphoreType.DMA,
        ],
        compiler_params=pltpu.CompilerParams(needs_layout_passes=False),
    )(_sc_hist_kernel)
    return f(x1, x2)


_NB = 4

def _ssim_kernel(x1_ref, x2_ref, m192_ref, m256t_ref, out_ref):
    @pl.when(pl.program_id(0) == 0)
    def _():
        out_ref[...] = jnp.zeros((1, 1), jnp.float32)

    m192 = m192_ref[...]
    m256t = m256t_ref[...]
    c1 = jnp.float32(0.01 ** 2)
    c2 = jnp.float32(0.03 ** 2)

    part = jnp.float32(0.0)
    for b in range(_NB):
        x1 = x1_ref[b]
        x2 = x2_ref[b]
        fields = jnp.concatenate(
            [x1, x2, x1 * x1, x2 * x2, x1 * x2], axis=1)
        v = jnp.dot(m192, fields, preferred_element_type=jnp.float32,
                    precision=lax.Precision.DEFAULT)
        convs = [
            jnp.dot(v[:, i * _W:(i + 1) * _W], m256t,
                    preferred_element_type=jnp.float32,
                    precision=lax.Precision.DEFAULT)
            for i in range(5)
        ]
        mu1, mu2, s1, s2, s12 = convs
        mu1_sq = mu1 * mu1
        mu2_sq = mu2 * mu2
        mu1_mu2 = mu1 * mu2
        sigma1_sq = s1 - mu1_sq
        sigma2_sq = s2 - mu2_sq
        sigma12 = s12 - mu1_mu2
        ssim_map = ((2 * mu1_mu2 + c1) * (2 * sigma12 + c2) /
                    ((mu1_sq + mu2_sq + c1) * (sigma1_sq + sigma2_sq + c2)))
        part = part + jnp.sum(ssim_map)

    out_ref[...] += jnp.full((1, 1), part, jnp.float32)


def _ssim_sum(x1, x2, m192, m256t):
    grid = _B // _NB
    return pl.pallas_call(
        _ssim_kernel,
        grid=(grid,),
        in_specs=[
            pl.BlockSpec((_NB, _H, _W), lambda i: (i, 0, 0)),
            pl.BlockSpec((_NB, _H, _W), lambda i: (i, 0, 0)),
            pl.BlockSpec((_H, _H), lambda i: (0, 0)),
            pl.BlockSpec((_W, _W), lambda i: (0, 0)),
        ],
        out_specs=pl.BlockSpec((1, 1), lambda i: (0, 0)),
        out_shape=jax.ShapeDtypeStruct((1, 1), jnp.float32),
        compiler_params=pltpu.CompilerParams(
            dimension_semantics=("arbitrary",)),
    )(x1, x2, m192, m256t)


def _final_kernel(hist_ref, ssim_ref, out_ref):
    cols = lax.broadcasted_iota(jnp.int32, (_B, _NBINP), 1)
    mask = cols < _NBIN
    neg = jnp.float32(-3.4e38)

    def absorb(h):
        rolled = jnp.concatenate(
            [h[:, 1:], jnp.zeros((_B, 1), jnp.float32)], axis=1)
        return h + jnp.where(cols == _NBIN - 1, rolled, 0.0)

    h1 = jnp.where(mask, absorb(hist_ref[:_B]), neg)
    h2 = jnp.where(mask, absorb(hist_ref[_B:]), neg)

    def softmax(h):
        m = jnp.max(h, axis=1, keepdims=True)
        e = jnp.exp(h - m)
        return e / jnp.sum(e, axis=1, keepdims=True)

    p1 = softmax(h1)
    p2 = softmax(h2)
    kld = jnp.sum(jnp.where(mask, jnp.exp(p2) * (p2 - p1), 0.0)) / _B

    ssim = ssim_ref[0, 0] / jnp.float32(_B * _H * _W)
    res = jnp.where(ssim > 0.75, kld + 1.0 - ssim, 1.0 - ssim)
    out_ref[...] = jnp.full((1, 1), res, jnp.float32)


def _final(hist, ssim_sum):
    return pl.pallas_call(
        _final_kernel,
        out_shape=jax.ShapeDtypeStruct((1, 1), jnp.float32),
    )(hist, ssim_sum)


def _banded(n, g):
    i = jnp.arange(n)[:, None]
    j = jnp.arange(n)[None, :]
    d = j - i + _PAD
    valid = (d >= 0) & (d <= _WIN - 1)
    return jnp.where(valid, g[jnp.clip(d, 0, _WIN - 1)], 0.0).astype(jnp.float32)


def kernel(img1, img2, window):
    x1 = img1.reshape(_B, _H, _W)
    x2 = img2.reshape(_B, _H, _W)

    w2 = window[0, 0]
    g = w2[_PAD] / jnp.sqrt(w2[_PAD, _PAD])
    m192 = _banded(_H, g)
    m256t = _banded(_W, g).T

    hist = _sc_hist(img1, img2)
    ssim_sum = _ssim_sum(x1, x2, m192, m256t)
    out = _final(hist, ssim_sum)
    return out[0, 0]
